# baseline (speedup 1.0000x reference)
import functools

import jax
import jax.numpy as jnp
from jax import lax
from jax.experimental import pallas as pl
from jax.experimental.pallas import tpu as pltpu
from jax.experimental.pallas import tpu_sc as plsc

N_NODES = 10000
N_EDGES = 320000
IN_CH = 128
OUT_CH = 128

NC, NS, LANES = 2, 16, 16
N_PAD = 10240
ROWS_PER_TILE = N_PAD // NS
CHUNK = 128
E_PAD = 327680
EPT = E_PAD // NS
EPT_DEG = E_PAD // (NC * NS)
ROW_BLK = 1024

_mesh = plsc.VectorSubcoreMesh(core_axis_name="c", subcore_axis_name="s")


@functools.partial(
    pl.kernel,
    out_type=jax.ShapeDtypeStruct((NC * N_PAD,), jnp.float32),
    mesh=_mesh,
    scratch_types=[
        pltpu.VMEM((EPT_DEG // CHUNK, CHUNK), jnp.int32),
        pltpu.VMEM((CHUNK,), jnp.float32),
        pltpu.VMEM_SHARED((N_PAD,), jnp.float32),
    ],
)
def _deg_kernel(dst2d_hbm, z1d_hbm, deg_out_hbm, dst_v, ones_v, deg_sh):
    c = lax.axis_index("c")
    s = lax.axis_index("s")
    t = c * NS + s
    pltpu.sync_copy(z1d_hbm, deg_sh.at[pl.ds(s * ROWS_PER_TILE, ROWS_PER_TILE)])
    pltpu.sync_copy(
        dst2d_hbm.at[pl.ds(t * (EPT_DEG // CHUNK), EPT_DEG // CHUNK)], dst_v
    )
    for i in range(CHUNK // LANES):
        ones_v[pl.ds(i * LANES, LANES)] = jnp.full((LANES,), 1.0, jnp.float32)
    plsc.subcore_barrier()

    def body(j, carry):
        pltpu.sync_copy(ones_v, deg_sh.at[dst_v.at[j]], add=True)
        return carry

    lax.fori_loop(0, EPT_DEG // CHUNK, body, 0)
    plsc.subcore_barrier()
    pltpu.sync_copy(
        deg_sh.at[pl.ds(s * ROWS_PER_TILE, ROWS_PER_TILE)],
        deg_out_hbm.at[pl.ds(c * N_PAD + s * ROWS_PER_TILE, ROWS_PER_TILE)],
    )


GRP = 16
N_GROUPS = EPT // (GRP * CHUNK)


def _make_seg_sum(split_edges):
    width = 128
    ept = EPT_DEG if split_edges else EPT

    n_chunks = ept // CHUNK

    @functools.partial(
        pl.kernel,
        out_type=jax.ShapeDtypeStruct((NC * N_PAD, width), jnp.float32),
        mesh=_mesh,
        scratch_types=[
            pltpu.VMEM((GRP * CHUNK,), jnp.int32),
            pltpu.VMEM((GRP, CHUNK), jnp.int32),
            pltpu.VMEM((CHUNK, width), jnp.float32),
            pltpu.VMEM((CHUNK, width), jnp.float32),
            pltpu.VMEM_SHARED((N_PAD, width), jnp.float32),
            pltpu.SemaphoreType.DMA,
            pltpu.SemaphoreType.DMA,
            pltpu.SemaphoreType.DMA,
            pltpu.SemaphoreType.DMA,
        ],
    )
    def seg(hs_hbm, src2_hbm, dst2d_hbm, zw_hbm, acc_out_hbm,
            src_v, dst_v, rows0, rows1, acc_sh, g0, g1, s0, s1):
        c = lax.axis_index("c")
        s = lax.axis_index("s")
        if split_edges:
            src_base = (c * NS + s) * ept
            dst_row_base = (c * NS + s) * n_chunks
        else:
            src_base = c * E_PAD + s * ept
            dst_row_base = s * n_chunks
        pltpu.sync_copy(zw_hbm, acc_sh.at[pl.ds(s * ROWS_PER_TILE, ROWS_PER_TILE)])
        plsc.subcore_barrier()

        def gather(j, buf, sem):
            return pltpu.make_async_copy(
                hs_hbm.at[src_v.at[pl.ds(j * CHUNK, CHUNK)]], buf, sem
            )

        def scatter(j, buf, sem):
            return pltpu.make_async_copy(buf, acc_sh.at[dst_v.at[j]], sem)

        def group(g, carry):
            pltpu.sync_copy(
                src2_hbm.at[pl.ds(src_base + g * GRP * CHUNK, GRP * CHUNK)], src_v)
            pltpu.sync_copy(dst2d_hbm.at[pl.ds(dst_row_base + g * GRP, GRP)],
                            dst_v)
            gather(0, rows0, g0).start()

            def body(k, carry2):
                j0 = 2 * k
                gather(j0, rows0, g0).wait()
                sc0 = scatter(j0, rows0, s0)
                sc0.start(add=True)

                @pl.when(k > 0)
                def _():
                    scatter(j0 - 1, rows1, s1).wait()
                gather(j0 + 1, rows1, g1).start()
                gather(j0 + 1, rows1, g1).wait()
                sc1 = scatter(j0 + 1, rows1, s1)
                sc1.start(add=True)
                sc0.wait()

                @pl.when(k < GRP // 2 - 1)
                def _():
                    gather(j0 + 2, rows0, g0).start()
                return carry2

            lax.fori_loop(0, GRP // 2, body, carry)
            scatter(GRP - 1, rows1, s1).wait()
            return carry

        lax.fori_loop(0, ept // (GRP * CHUNK), group, 0)
        plsc.subcore_barrier()
        pltpu.sync_copy(
            acc_sh.at[pl.ds(s * ROWS_PER_TILE, ROWS_PER_TILE)],
            acc_out_hbm.at[pl.ds(c * N_PAD + s * ROWS_PER_TILE, ROWS_PER_TILE)],
        )

    return seg


_seg_sum_feat = _make_seg_sum(split_edges=False)
_seg_sum_part = _make_seg_sum(split_edges=True)


def _mm1_body(x_ref, w1_ref, deg_ref, hs_ref, dis_ref):
    deg = deg_ref[0] + deg_ref[1] + 1.0
    dis = lax.rsqrt(deg)
    dis_ref[...] = dis
    h = jnp.dot(x_ref[...], w1_ref[...], preferred_element_type=jnp.float32)
    hs = h * dis[:, None]
    hs_ref[0] = hs[:, :128]
    hs_ref[1] = hs[:, 128:]


def _mm2_body(acc_ref, hs_ref, dis_ref, b1_ref, w2_ref, hs2_ref):
    dis = dis_ref[...]
    b1 = b1_ref[...]
    h0 = jax.nn.relu(dis[:, None] * (acc_ref[0] + hs_ref[0]) + b1[None, :128])
    h1 = jax.nn.relu(dis[:, None] * (acc_ref[1] + hs_ref[1]) + b1[None, 128:])
    h = jnp.concatenate([h0, h1], axis=1)
    hs2 = jnp.dot(h, w2_ref[...], preferred_element_type=jnp.float32)
    hs2_ref[...] = hs2 * dis[:, None]


def _fin_body(acc_ref, hs_ref, dis_ref, b2_ref, out_ref):
    dis = dis_ref[...]
    acc = acc_ref[0] + acc_ref[1]
    out_ref[...] = dis[:, None] * (acc + hs_ref[...]) + b2_ref[...][None, :]


def _row_grid():
    return N_PAD // ROW_BLK


def _tc_mm1(x_pad, W1, deg2):
    return pl.pallas_call(
        _mm1_body,
        grid=(_row_grid(),),
        in_specs=[
            pl.BlockSpec((ROW_BLK, IN_CH), lambda i: (i, 0)),
            pl.BlockSpec((IN_CH, 256), lambda i: (0, 0)),
            pl.BlockSpec((2, ROW_BLK), lambda i: (0, i)),
        ],
        out_specs=[
            pl.BlockSpec((2, ROW_BLK, 128), lambda i: (0, i, 0)),
            pl.BlockSpec((ROW_BLK,), lambda i: (i,)),
        ],
        out_shape=[
            jax.ShapeDtypeStruct((2, N_PAD, 128), jnp.float32),
            jax.ShapeDtypeStruct((N_PAD,), jnp.float32),
        ],
    )(x_pad, W1, deg2)


def _tc_mm2(acc1, hs1, dis, b1, W2):
    return pl.pallas_call(
        _mm2_body,
        grid=(_row_grid(),),
        in_specs=[
            pl.BlockSpec((2, ROW_BLK, 128), lambda i: (0, i, 0)),
            pl.BlockSpec((2, ROW_BLK, 128), lambda i: (0, i, 0)),
            pl.BlockSpec((ROW_BLK,), lambda i: (i,)),
            pl.BlockSpec((256,), lambda i: (0,)),
            pl.BlockSpec((256, 128), lambda i: (0, 0)),
        ],
        out_specs=pl.BlockSpec((ROW_BLK, 128), lambda i: (i, 0)),
        out_shape=jax.ShapeDtypeStruct((N_PAD, 128), jnp.float32),
    )(acc1, hs1, dis, b1, W2)


def _tc_fin(acc2, hs2, dis, b2):
    return pl.pallas_call(
        _fin_body,
        grid=(_row_grid(),),
        in_specs=[
            pl.BlockSpec((2, ROW_BLK, 128), lambda i: (0, i, 0)),
            pl.BlockSpec((ROW_BLK, 128), lambda i: (i, 0)),
            pl.BlockSpec((ROW_BLK,), lambda i: (i,)),
            pl.BlockSpec((OUT_CH,), lambda i: (0,)),
        ],
        out_specs=pl.BlockSpec((ROW_BLK, OUT_CH), lambda i: (i, 0)),
        out_shape=jax.ShapeDtypeStruct((N_PAD, OUT_CH), jnp.float32),
    )(acc2, hs2, dis, b2)


def kernel(x, edge_index, W1, b1, W2, b2):
    ei = edge_index.astype(jnp.int32)
    pad_e = E_PAD - N_EDGES
    src = jnp.concatenate([ei[0], jnp.full((pad_e,), N_NODES, jnp.int32)])
    dst = jnp.concatenate([ei[1], jnp.full((pad_e,), N_NODES, jnp.int32)])
    src2 = jnp.concatenate([src, src + N_PAD])
    dst2d = dst.reshape(E_PAD // CHUNK, CHUNK)

    x_pad = jnp.pad(x, ((0, N_PAD - N_NODES), (0, 0)))
    z1d = jnp.zeros((ROWS_PER_TILE,), jnp.float32)
    z128 = jnp.zeros((ROWS_PER_TILE, 128), jnp.float32)

    deg2 = _deg_kernel(dst2d, z1d).reshape(2, N_PAD)

    hs1, dis = _tc_mm1(x_pad, W1, deg2)
    acc1 = _seg_sum_feat(hs1.reshape(NC * N_PAD, 128), src2, dst2d, z128)
    acc1 = acc1.reshape(2, N_PAD, 128)

    hs2 = _tc_mm2(acc1, hs1, dis, b1, W2)
    acc2 = _seg_sum_part(hs2, src, dst2d, z128)
    acc2 = acc2.reshape(2, N_PAD, 128)

    out = _tc_fin(acc2, hs2, dis, b2)
    return out[:N_NODES]


# speedup vs baseline: 1.0176x; 1.0176x over previous
import functools

import jax
import jax.numpy as jnp
from jax import lax
from jax.experimental import pallas as pl
from jax.experimental.pallas import tpu as pltpu
from jax.experimental.pallas import tpu_sc as plsc

N_NODES = 10000
N_EDGES = 320000
IN_CH = 128
OUT_CH = 128

NC, NS, LANES = 2, 16, 16
N_PAD = 10240
ROWS_PER_TILE = N_PAD // NS
CHUNK = 64
NSLOT = 4
E_PAD = 327680
EPT = E_PAD // NS
EPT_DEG = E_PAD // (NC * NS)
GRP = 32
ROW_BLK = 1024

_mesh = plsc.VectorSubcoreMesh(core_axis_name="c", subcore_axis_name="s")


@functools.partial(
    pl.kernel,
    out_type=jax.ShapeDtypeStruct((NC * N_PAD,), jnp.float32),
    mesh=_mesh,
    scratch_types=[
        pltpu.VMEM((EPT_DEG // 128, 128), jnp.int32),
        pltpu.VMEM((128,), jnp.float32),
        pltpu.VMEM_SHARED((N_PAD,), jnp.float32),
    ],
)
def _deg_kernel(dst2d_hbm, z1d_hbm, deg_out_hbm, dst_v, ones_v, deg_sh):
    c = lax.axis_index("c")
    s = lax.axis_index("s")
    t = c * NS + s
    pltpu.sync_copy(z1d_hbm, deg_sh.at[pl.ds(s * ROWS_PER_TILE, ROWS_PER_TILE)])
    pltpu.sync_copy(
        dst2d_hbm.at[pl.ds(t * (EPT_DEG // 128), EPT_DEG // 128)], dst_v
    )
    for i in range(128 // LANES):
        ones_v[pl.ds(i * LANES, LANES)] = jnp.full((LANES,), 1.0, jnp.float32)
    plsc.subcore_barrier()

    def body(j, carry):
        pltpu.sync_copy(ones_v, deg_sh.at[dst_v.at[j]], add=True)
        return carry

    lax.fori_loop(0, EPT_DEG // 128, body, 0)
    plsc.subcore_barrier()
    pltpu.sync_copy(
        deg_sh.at[pl.ds(s * ROWS_PER_TILE, ROWS_PER_TILE)],
        deg_out_hbm.at[pl.ds(c * N_PAD + s * ROWS_PER_TILE, ROWS_PER_TILE)],
    )


def _make_seg_sum(split_edges):
    width = 128
    ept = EPT_DEG if split_edges else EPT
    n_chunks = ept // CHUNK
    n_groups = n_chunks // GRP
    rounds = GRP // NSLOT

    @functools.partial(
        pl.kernel,
        out_type=jax.ShapeDtypeStruct((NC * N_PAD, width), jnp.float32),
        mesh=_mesh,
        scratch_types=[
            pltpu.VMEM((GRP * CHUNK,), jnp.int32),
            pltpu.VMEM((GRP * CHUNK // 128, 128), jnp.int32),
            [pltpu.VMEM((CHUNK, width), jnp.float32) for _ in range(NSLOT)],
            pltpu.VMEM_SHARED((N_PAD, width), jnp.float32),
            [pltpu.SemaphoreType.DMA for _ in range(NSLOT)],
            [pltpu.SemaphoreType.DMA for _ in range(NSLOT)],
        ],
    )
    def seg(hs_hbm, src2_hbm, dst2d_hbm, zw_hbm, acc_out_hbm,
            src_v, dst_v, rows, acc_sh, gsem, ssem):
        c = lax.axis_index("c")
        s = lax.axis_index("s")
        if split_edges:
            src_base = (c * NS + s) * ept
            dst_row_base = (c * NS + s) * (ept // 128)
        else:
            src_base = c * E_PAD + s * ept
            dst_row_base = s * (ept // 128)
        pltpu.sync_copy(zw_hbm, acc_sh.at[pl.ds(s * ROWS_PER_TILE, ROWS_PER_TILE)])
        plsc.subcore_barrier()

        def gather(j, b):
            return pltpu.make_async_copy(
                hs_hbm.at[src_v.at[pl.ds(j * CHUNK, CHUNK)]], rows[b], gsem[b]
            )

        def scatter(j, b):
            return pltpu.make_async_copy(
                rows[b],
                acc_sh.at[dst_v.at[j // 2, pl.ds((j % 2) * CHUNK, CHUNK)]],
                ssem[b],
            )

        def group(g, carry):
            pltpu.sync_copy(
                src2_hbm.at[pl.ds(src_base + g * GRP * CHUNK, GRP * CHUNK)], src_v)
            pltpu.sync_copy(
                dst2d_hbm.at[pl.ds(dst_row_base + g * (GRP * CHUNK // 128),
                                   GRP * CHUNK // 128)], dst_v)
            for b in range(NSLOT):
                gather(b, b).start()

            def body(k, carry2):
                j0 = k * NSLOT
                for b in range(NSLOT):
                    gather(j0 + b, b).wait()
                    scatter(j0 + b, b).start(add=True)

                @pl.when(k < rounds - 1)
                def _():
                    for b in range(NSLOT):
                        scatter(j0 + b, b).wait()
                        gather(j0 + NSLOT + b, b).start()
                return carry2

            lax.fori_loop(0, rounds, body, carry)
            for b in range(NSLOT):
                scatter(GRP - NSLOT + b, b).wait()
            return carry

        lax.fori_loop(0, n_groups, group, 0)
        plsc.subcore_barrier()
        pltpu.sync_copy(
            acc_sh.at[pl.ds(s * ROWS_PER_TILE, ROWS_PER_TILE)],
            acc_out_hbm.at[pl.ds(c * N_PAD + s * ROWS_PER_TILE, ROWS_PER_TILE)],
        )

    return seg


_seg_sum_feat = _make_seg_sum(split_edges=False)
_seg_sum_part = _make_seg_sum(split_edges=True)


def _mm1_body(x_ref, w1_ref, deg_ref, hs_ref, dis_ref):
    deg = deg_ref[0] + deg_ref[1] + 1.0
    dis = lax.rsqrt(deg)
    dis_ref[...] = dis
    h = jnp.dot(x_ref[...], w1_ref[...], preferred_element_type=jnp.float32)
    hs = h * dis[:, None]
    hs_ref[0] = hs[:, :128]
    hs_ref[1] = hs[:, 128:]


def _mm2_body(acc_ref, hs_ref, dis_ref, b1_ref, w2_ref, hs2_ref):
    dis = dis_ref[...]
    b1 = b1_ref[...]
    h0 = jax.nn.relu(dis[:, None] * (acc_ref[0] + hs_ref[0]) + b1[None, :128])
    h1 = jax.nn.relu(dis[:, None] * (acc_ref[1] + hs_ref[1]) + b1[None, 128:])
    h = jnp.concatenate([h0, h1], axis=1)
    hs2 = jnp.dot(h, w2_ref[...], preferred_element_type=jnp.float32)
    hs2_ref[...] = hs2 * dis[:, None]


def _fin_body(acc_ref, hs_ref, dis_ref, b2_ref, out_ref):
    dis = dis_ref[...]
    acc = acc_ref[0] + acc_ref[1]
    out_ref[...] = dis[:, None] * (acc + hs_ref[...]) + b2_ref[...][None, :]


def _row_grid():
    return N_PAD // ROW_BLK


def _tc_mm1(x_pad, W1, deg2):
    return pl.pallas_call(
        _mm1_body,
        grid=(_row_grid(),),
        in_specs=[
            pl.BlockSpec((ROW_BLK, IN_CH), lambda i: (i, 0)),
            pl.BlockSpec((IN_CH, 256), lambda i: (0, 0)),
            pl.BlockSpec((2, ROW_BLK), lambda i: (0, i)),
        ],
        out_specs=[
            pl.BlockSpec((2, ROW_BLK, 128), lambda i: (0, i, 0)),
            pl.BlockSpec((ROW_BLK,), lambda i: (i,)),
        ],
        out_shape=[
            jax.ShapeDtypeStruct((2, N_PAD, 128), jnp.float32),
            jax.ShapeDtypeStruct((N_PAD,), jnp.float32),
        ],
    )(x_pad, W1, deg2)


def _tc_mm2(acc1, hs1, dis, b1, W2):
    return pl.pallas_call(
        _mm2_body,
        grid=(_row_grid(),),
        in_specs=[
            pl.BlockSpec((2, ROW_BLK, 128), lambda i: (0, i, 0)),
            pl.BlockSpec((2, ROW_BLK, 128), lambda i: (0, i, 0)),
            pl.BlockSpec((ROW_BLK,), lambda i: (i,)),
            pl.BlockSpec((256,), lambda i: (0,)),
            pl.BlockSpec((256, 128), lambda i: (0, 0)),
        ],
        out_specs=pl.BlockSpec((ROW_BLK, 128), lambda i: (i, 0)),
        out_shape=jax.ShapeDtypeStruct((N_PAD, 128), jnp.float32),
    )(acc1, hs1, dis, b1, W2)


def _tc_fin(acc2, hs2, dis, b2):
    return pl.pallas_call(
        _fin_body,
        grid=(_row_grid(),),
        in_specs=[
            pl.BlockSpec((2, ROW_BLK, 128), lambda i: (0, i, 0)),
            pl.BlockSpec((ROW_BLK, 128), lambda i: (i, 0)),
            pl.BlockSpec((ROW_BLK,), lambda i: (i,)),
            pl.BlockSpec((OUT_CH,), lambda i: (0,)),
        ],
        out_specs=pl.BlockSpec((ROW_BLK, OUT_CH), lambda i: (i, 0)),
        out_shape=jax.ShapeDtypeStruct((N_PAD, OUT_CH), jnp.float32),
    )(acc2, hs2, dis, b2)


def kernel(x, edge_index, W1, b1, W2, b2):
    ei = edge_index.astype(jnp.int32)
    pad_e = E_PAD - N_EDGES
    src = jnp.concatenate([ei[0], jnp.full((pad_e,), N_NODES, jnp.int32)])
    dst = jnp.concatenate([ei[1], jnp.full((pad_e,), N_NODES, jnp.int32)])
    src2 = jnp.concatenate([src, src + N_PAD])
    dst2d = dst.reshape(E_PAD // 128, 128)

    x_pad = jnp.pad(x, ((0, N_PAD - N_NODES), (0, 0)))
    z1d = jnp.zeros((ROWS_PER_TILE,), jnp.float32)
    z128 = jnp.zeros((ROWS_PER_TILE, 128), jnp.float32)

    deg2 = _deg_kernel(dst2d, z1d).reshape(2, N_PAD)

    hs1, dis = _tc_mm1(x_pad, W1, deg2)
    acc1 = _seg_sum_feat(hs1.reshape(NC * N_PAD, 128), src2, dst2d, z128)
    acc1 = acc1.reshape(2, N_PAD, 128)

    hs2 = _tc_mm2(acc1, hs1, dis, b1, W2)
    acc2 = _seg_sum_part(hs2, src, dst2d, z128)
    acc2 = acc2.reshape(2, N_PAD, 128)

    out = _tc_fin(acc2, hs2, dis, b2)
    return out[:N_NODES]
